# baseline (speedup 1.0000x reference)
import functools

import jax
import jax.numpy as jnp
from jax.experimental import pallas as pl
from jax.experimental.pallas import tpu as pltpu

_VMEM_LIMIT = 48 * 1024 * 1024


def _sigmoid(x):
    return pl.reciprocal(1.0 + jnp.exp(-x), approx=True)


def _cbam_kernel(x_ref, wt_ref, b_ref, k_ref, o_ref, *, inv_m, inv_c):
    x = x_ref[0]
    c, m = x.shape

    ones_m = jnp.ones((m, 8), jnp.float32)
    mean8 = jnp.dot(x, ones_m, preferred_element_type=jnp.float32) * inv_m

    z8 = jnp.dot(wt_ref[...], mean8, preferred_element_type=jnp.float32)
    ca8 = _sigmoid(z8 + b_ref[...])
    ca = ca8[:, 0:1]

    y = x * ca

    ones_c = jnp.ones((8, c), jnp.float32)
    meanm = jnp.dot(ones_c, y, preferred_element_type=jnp.float32)[0:1] * inv_c
    maxm = jnp.max(y, axis=0, keepdims=True)

    mm = jnp.concatenate([meanm, maxm], axis=1)
    mm8 = jnp.broadcast_to(mm, (8, 2 * m))
    conv = jnp.dot(mm8, k_ref[...], preferred_element_type=jnp.float32)
    sa = _sigmoid(conv[0:1])

    o_ref[0] = y * sa


def _build_conv_matrix(w_sa, h, w):
    m = h * w
    pi = jnp.arange(m, dtype=jnp.int32)
    qi = jnp.arange(m, dtype=jnp.int32)
    dy = pi[:, None] // w - qi[None, :] // w + 3
    dx = pi[:, None] % w - qi[None, :] % w + 3
    valid = (dy >= 0) & (dy <= 6) & (dx >= 0) & (dx <= 6)
    idx = jnp.clip(dy * 7 + dx, 0, 48)
    k0 = jnp.where(valid, w_sa[0][idx], 0.0)
    k1 = jnp.where(valid, w_sa[1][idx], 0.0)
    return jnp.concatenate([k0, k1], axis=0)


def kernel(x, w_fc, b_fc, w_sa):
    B, C, H, W = x.shape
    M = H * W
    x2d = x.reshape(B, C, M)
    wt = w_fc.T
    b_col = b_fc.reshape(C, 1)
    K = _build_conv_matrix(w_sa, H, W)

    body = functools.partial(
        _cbam_kernel, inv_m=1.0 / float(M), inv_c=1.0 / float(C))
    out2d = pl.pallas_call(
        body,
        out_shape=jax.ShapeDtypeStruct((B, C, M), x.dtype),
        grid=(B,),
        in_specs=[
            pl.BlockSpec((1, C, M), lambda b: (b, 0, 0)),
            pl.BlockSpec((C, C), lambda b: (0, 0)),
            pl.BlockSpec((C, 1), lambda b: (0, 0)),
            pl.BlockSpec((2 * M, M), lambda b: (0, 0)),
        ],
        out_specs=pl.BlockSpec((1, C, M), lambda b: (b, 0, 0)),
        compiler_params=pltpu.CompilerParams(
            dimension_semantics=("parallel",),
            vmem_limit_bytes=_VMEM_LIMIT),
    )(x2d, wt, b_col, K)
    return out2d.reshape(B, C, H, W)


# speedup vs baseline: 1.0614x; 1.0614x over previous
import functools

import jax
import jax.numpy as jnp
from jax.experimental import pallas as pl
from jax.experimental.pallas import tpu as pltpu

_VMEM_LIMIT = 48 * 1024 * 1024


def _sigmoid(x):
    return pl.reciprocal(1.0 + jnp.exp(-x), approx=True)


def _cbam_kernel(x_ref, w_ref, b_ref, k_ref, o_ref, *, inv_m, inv_c):
    x = x_ref[0]
    c, m = x.shape

    ones_m = jnp.ones((m, 8), jnp.float32)
    mean8 = jnp.dot(x, ones_m, preferred_element_type=jnp.float32) * inv_m

    z8 = jax.lax.dot_general(
        w_ref[...], mean8, (((0,), (0,)), ((), ())),
        preferred_element_type=jnp.float32)
    ca8 = _sigmoid(z8 + b_ref[...])
    ca = ca8[:, 0:1]

    y = x * ca

    ones_c = jnp.ones((8, c), jnp.float32)
    meanm = jnp.dot(ones_c, y, preferred_element_type=jnp.float32)[0:1] * inv_c
    maxm = jnp.max(y, axis=0, keepdims=True)

    mm = jnp.concatenate([meanm, maxm], axis=1)
    mm8 = jnp.broadcast_to(mm, (8, 2 * m))
    conv = jnp.dot(mm8, k_ref[...], preferred_element_type=jnp.float32)
    sa = _sigmoid(conv[0:1])

    o_ref[0] = y * sa


def _build_conv_matrix(w_sa, h, w):
    taps = jnp.arange(7, dtype=jnp.int32)[:, None, None]
    hy = jnp.arange(h, dtype=jnp.int32)
    wx = jnp.arange(w, dtype=jnp.int32)
    by = (hy[None, :, None] - hy[None, None, :] + 3 == taps).astype(jnp.float32)
    bx = (wx[None, :, None] - wx[None, None, :] + 3 == taps).astype(jnp.float32)
    w3 = w_sa.reshape(2, 7, 7)
    t = jnp.einsum('kij,iab->kjab', w3, by)
    k5 = (t[:, :, :, None, :, None] * bx[None, :, None, :, None, :]).sum(axis=1)
    return k5.reshape(2 * h * w, h * w)


def kernel(x, w_fc, b_fc, w_sa):
    B, C, H, W = x.shape
    M = H * W
    x2d = x.reshape(B, C, M)
    b_col = b_fc.reshape(C, 1)
    K = _build_conv_matrix(w_sa, H, W)

    body = functools.partial(
        _cbam_kernel, inv_m=1.0 / float(M), inv_c=1.0 / float(C))
    out2d = pl.pallas_call(
        body,
        out_shape=jax.ShapeDtypeStruct((B, C, M), x.dtype),
        grid=(B,),
        in_specs=[
            pl.BlockSpec((1, C, M), lambda b: (b, 0, 0)),
            pl.BlockSpec((C, C), lambda b: (0, 0)),
            pl.BlockSpec((C, 1), lambda b: (0, 0)),
            pl.BlockSpec((2 * M, M), lambda b: (0, 0)),
        ],
        out_specs=pl.BlockSpec((1, C, M), lambda b: (b, 0, 0)),
        compiler_params=pltpu.CompilerParams(
            dimension_semantics=("parallel",),
            vmem_limit_bytes=_VMEM_LIMIT),
    )(x2d, w_fc, b_col, K)
    return out2d.reshape(B, C, H, W)


# speedup vs baseline: 1.0636x; 1.0021x over previous
import functools

import jax
import jax.numpy as jnp
from jax.experimental import pallas as pl
from jax.experimental.pallas import tpu as pltpu

_VMEM_LIMIT = 48 * 1024 * 1024


def _sigmoid(x):
    return pl.reciprocal(1.0 + jnp.exp(-x), approx=True)


def _cbam_kernel(x_ref, w_ref, b_ref, k_ref, o_ref, *, inv_m, inv_c):
    x = x_ref[0]
    c, m = x.shape

    ones_m = jnp.ones((m, 8), jnp.float32)
    mean8 = jnp.dot(x, ones_m, preferred_element_type=jnp.float32) * inv_m

    z8 = jax.lax.dot_general(
        w_ref[...], mean8, (((0,), (0,)), ((), ())),
        preferred_element_type=jnp.float32)
    ca8 = _sigmoid(z8 + b_ref[...])
    ca = ca8[:, 0:1]

    y = x * ca

    ones_c = jnp.ones((8, c), jnp.float32)
    meanm = jnp.dot(ones_c, y, preferred_element_type=jnp.float32)[0:1] * inv_c
    maxm = jnp.max(y, axis=0, keepdims=True)

    mm = jnp.concatenate([meanm, maxm], axis=1)
    mm8 = jnp.broadcast_to(mm, (8, 2 * m))
    conv = jnp.dot(mm8, k_ref[...], preferred_element_type=jnp.float32)
    sa = _sigmoid(conv[0:1])

    o_ref[0] = y * sa


def _build_conv_matrix(w_sa, h, w):
    taps = jnp.arange(7, dtype=jnp.int32)[:, None, None]
    hy = jnp.arange(h, dtype=jnp.int32)
    wx = jnp.arange(w, dtype=jnp.int32)
    by = (hy[None, :, None] - hy[None, None, :] + 3 == taps).astype(jnp.float32)
    bx = (wx[None, :, None] - wx[None, None, :] + 3 == taps).astype(jnp.float32)
    w3 = w_sa.reshape(2, 7, 7)
    t = jnp.einsum('kij,iab->kjab', w3, by)
    k5 = (t[:, :, :, None, :, None] * bx[None, :, None, :, None, :]).sum(axis=1)
    return k5.reshape(2 * h * w, h * w)


def kernel(x, w_fc, b_fc, w_sa):
    B, C, H, W = x.shape
    M = H * W
    x2d = x.reshape(B, C, M)
    b_col = b_fc.reshape(C, 1)
    K = _build_conv_matrix(w_sa, H, W)

    body = functools.partial(
        _cbam_kernel, inv_m=1.0 / float(M), inv_c=1.0 / float(C))
    nb = B // 2
    out2d = pl.pallas_call(
        body,
        out_shape=jax.ShapeDtypeStruct((B, C, M), x.dtype),
        grid=(2, nb),
        in_specs=[
            pl.BlockSpec((1, C, M), lambda c, t: (c * nb + t, 0, 0)),
            pl.BlockSpec((C, C), lambda c, t: (0, 0)),
            pl.BlockSpec((C, 1), lambda c, t: (0, 0)),
            pl.BlockSpec((2 * M, M), lambda c, t: (0, 0)),
        ],
        out_specs=pl.BlockSpec((1, C, M), lambda c, t: (c * nb + t, 0, 0)),
        compiler_params=pltpu.CompilerParams(
            dimension_semantics=("parallel", "arbitrary"),
            vmem_limit_bytes=_VMEM_LIMIT),
    )(x2d, w_fc, b_col, K)
    return out2d.reshape(B, C, H, W)


# speedup vs baseline: 1.4280x; 1.3426x over previous
import functools

import jax
import jax.numpy as jnp
from jax.experimental import pallas as pl
from jax.experimental.pallas import tpu as pltpu

_VMEM_LIMIT = 48 * 1024 * 1024


def _sigmoid(x):
    return pl.reciprocal(1.0 + jnp.exp(-x), approx=True)


def _cbam_kernel(x_ref, w_ref, b_ref, kt_ref, o_ref, *, inv_m, inv_c):
    x = x_ref[0]
    m, c = x.shape

    ones_m = jnp.ones((8, m), jnp.float32)
    mean8 = jnp.dot(ones_m, x, preferred_element_type=jnp.float32) * inv_m

    z8 = jnp.dot(mean8, w_ref[...], preferred_element_type=jnp.float32)
    ca = _sigmoid(z8 + b_ref[...])[0:1]

    y = x * ca

    ones_c = jnp.ones((c, 8), jnp.float32)
    meanm8 = jnp.dot(y, ones_c, preferred_element_type=jnp.float32) * inv_c
    maxm8 = jnp.broadcast_to(jnp.max(y, axis=1, keepdims=True), (m, 8))
    mm8 = jnp.concatenate([meanm8, maxm8], axis=0)

    conv8 = jnp.dot(kt_ref[...], mm8, preferred_element_type=jnp.float32)
    sa = _sigmoid(conv8[:, 0:1])

    o_ref[0] = y * sa


def _build_conv_matrix(w_sa, h, w):
    taps = jnp.arange(7, dtype=jnp.int32)[:, None, None]
    hy = jnp.arange(h, dtype=jnp.int32)
    wx = jnp.arange(w, dtype=jnp.int32)
    by = (hy[None, :, None] - hy[None, None, :] + 3 == taps).astype(jnp.float32)
    bx = (wx[None, :, None] - wx[None, None, :] + 3 == taps).astype(jnp.float32)
    w3 = w_sa.reshape(2, 7, 7)
    u = jnp.einsum('kij,iab->kjab', w3, by)
    ut = u.transpose(3, 0, 1, 2)[:, None, :, :, :, None]
    bxt = bx.transpose(2, 0, 1)[None, :, None, :, None, :]
    kt = (ut * bxt).sum(axis=3)
    return kt.reshape(h * w, 2 * h * w)


def kernel(x, w_fc, b_fc, w_sa):
    B, C, H, W = x.shape
    M = H * W
    xh = jnp.transpose(x, (0, 2, 3, 1)).reshape(B, M, C)
    Kt = _build_conv_matrix(w_sa, H, W)

    body = functools.partial(
        _cbam_kernel, inv_m=1.0 / float(M), inv_c=1.0 / float(C))
    nb = B // 2
    outh = pl.pallas_call(
        body,
        out_shape=jax.ShapeDtypeStruct((B, M, C), x.dtype),
        grid=(2, nb),
        in_specs=[
            pl.BlockSpec((1, M, C), lambda ci, t: (ci * nb + t, 0, 0)),
            pl.BlockSpec((C, C), lambda ci, t: (0, 0)),
            pl.BlockSpec((1, C), lambda ci, t: (0, 0)),
            pl.BlockSpec((M, 2 * M), lambda ci, t: (0, 0)),
        ],
        out_specs=pl.BlockSpec((1, M, C), lambda ci, t: (ci * nb + t, 0, 0)),
        compiler_params=pltpu.CompilerParams(
            dimension_semantics=("parallel", "arbitrary"),
            vmem_limit_bytes=_VMEM_LIMIT),
    )(xh, w_fc, b_fc, Kt)
    return jnp.transpose(outh.reshape(B, H, W, C), (0, 3, 1, 2))


# speedup vs baseline: 2.9644x; 2.0760x over previous
import functools

import jax
import jax.numpy as jnp
from jax.experimental import pallas as pl
from jax.experimental.pallas import tpu as pltpu

_VMEM_LIMIT = 48 * 1024 * 1024


def _sigmoid(x):
    return pl.reciprocal(1.0 + jnp.exp(-x), approx=True)


def _cbam_kernel(x_ref, w_ref, b_ref, xw_ref, sy_ref, selh_ref, selht_ref,
                 wm_ref, o_ref, *, inv_m, inv_c, h, w):
    x = x_ref[0]
    m, c = x.shape

    ones_m = jnp.ones((8, m), jnp.float32)
    mean8 = jnp.dot(ones_m, x, preferred_element_type=jnp.float32) * inv_m
    z8 = jnp.dot(mean8, w_ref[...], preferred_element_type=jnp.float32)
    ca = _sigmoid(z8 + b_ref[...])[0:1]

    y = x * ca

    ones_c = jnp.ones((c, w), jnp.float32)
    meanm = jnp.dot(y, ones_c, preferred_element_type=jnp.float32) * inv_c
    maxm = jnp.broadcast_to(jnp.max(y, axis=1, keepdims=True), (m, w))
    yb = jnp.concatenate([meanm, maxm], axis=1) * wm_ref[...]

    zz = jnp.dot(selh_ref[...], yb, preferred_element_type=jnp.float32)
    s1 = jnp.dot(zz, xw_ref[...], preferred_element_type=jnp.float32)
    conv = jnp.zeros((h, w), jnp.float32)
    for i in range(7):
        conv = conv + jnp.dot(
            sy_ref[i * h:(i + 1) * h, :], s1[:, i * w:(i + 1) * w],
            preferred_element_type=jnp.float32)
    sa2 = _sigmoid(conv)

    tmp = jnp.dot(selht_ref[...], sa2, preferred_element_type=jnp.float32)
    sacol = jnp.sum(tmp * wm_ref[:, 0:w], axis=1, keepdims=True)

    o_ref[0] = y * sacol


def _build_consts(w_sa, h, w):
    f32 = jnp.float32
    w3 = w_sa.reshape(2, 7, 7)
    pw = jnp.arange(w, dtype=jnp.int32)
    dx = pw[:, None] - pw[None, :] + 3
    band = (dx[None] == jnp.arange(7, dtype=jnp.int32)[:, None, None])
    bandf = band.astype(f32)
    xw = jnp.einsum('kij,jab->kiab', w3, bandf)
    xwcat = xw.transpose(0, 2, 1, 3).reshape(2 * w, 7 * w)

    hy = jnp.arange(h, dtype=jnp.int32)
    dy = hy[:, None] - hy[None, :] + 3
    sy = (dy[None] == jnp.arange(7, dtype=jnp.int32)[:, None, None])
    syf = sy.transpose(0, 2, 1).astype(f32).reshape(7 * h, h)

    p = jnp.arange(h * w, dtype=jnp.int32)
    selht = (p[:, None] // w == hy[None, :]).astype(f32)
    selh = selht.T
    w2 = jnp.arange(2 * w, dtype=jnp.int32)
    wm = (p[:, None] % w == w2[None, :] % w).astype(f32)
    return xwcat, syf, selh, selht, wm


def kernel(x, w_fc, b_fc, w_sa):
    B, C, H, W = x.shape
    M = H * W
    xh = jnp.transpose(x, (0, 2, 3, 1)).reshape(B, M, C)
    xwcat, syf, selh, selht, wm = _build_consts(w_sa, H, W)

    body = functools.partial(
        _cbam_kernel, inv_m=1.0 / float(M), inv_c=1.0 / float(C), h=H, w=W)
    nb = B // 2
    cspec = lambda shape: pl.BlockSpec(shape, lambda ci, t: tuple(0 for _ in shape))
    outh = pl.pallas_call(
        body,
        out_shape=jax.ShapeDtypeStruct((B, M, C), x.dtype),
        grid=(2, nb),
        in_specs=[
            pl.BlockSpec((1, M, C), lambda ci, t: (ci * nb + t, 0, 0)),
            cspec((C, C)),
            cspec((1, C)),
            cspec((2 * W, 7 * W)),
            cspec((7 * H, H)),
            cspec((H, M)),
            cspec((M, H)),
            cspec((M, 2 * W)),
        ],
        out_specs=pl.BlockSpec((1, M, C), lambda ci, t: (ci * nb + t, 0, 0)),
        compiler_params=pltpu.CompilerParams(
            dimension_semantics=("parallel", "arbitrary"),
            vmem_limit_bytes=_VMEM_LIMIT),
    )(xh, w_fc, b_fc, xwcat, syf, selh, selht, wm)
    return jnp.transpose(outh.reshape(B, H, W, C), (0, 3, 1, 2))


# speedup vs baseline: 3.7185x; 1.2544x over previous
import functools

import jax
import jax.numpy as jnp
from jax.experimental import pallas as pl
from jax.experimental.pallas import tpu as pltpu

_VMEM_LIMIT = 48 * 1024 * 1024


def _sigmoid(x):
    return pl.reciprocal(1.0 + jnp.exp(-x), approx=True)


def _one_batch(x_ref, w_ref, b_ref, xw_ref, sy_ref, selh_ref, selht_ref,
               wm_ref, o_ref, g, inv_m, inv_c, h, w):
    x = x_ref[g]
    m, c = x.shape

    ones_m = jnp.full((8, m), inv_m, jnp.float32)
    mean8 = jnp.dot(ones_m, x, preferred_element_type=jnp.float32)
    z8 = jnp.dot(mean8, w_ref[...], preferred_element_type=jnp.float32)
    ca = _sigmoid(z8 + b_ref[...])[0:1]

    y = x * ca

    ones_c = jnp.full((c, w), inv_c, jnp.float32)
    meanm = jnp.dot(y, ones_c, preferred_element_type=jnp.float32)
    maxm = jnp.broadcast_to(jnp.max(y, axis=1, keepdims=True), (m, w))
    yb = jnp.concatenate([meanm, maxm], axis=1) * wm_ref[...]

    zz = jnp.dot(selh_ref[...], yb, preferred_element_type=jnp.float32)
    s1 = jnp.dot(zz, xw_ref[...], preferred_element_type=jnp.float32)
    parts = [
        jnp.dot(sy_ref[i * h:(i + 1) * h, :], s1[:, i * w:(i + 1) * w],
                preferred_element_type=jnp.float32)
        for i in range(7)
    ]
    conv = ((parts[0] + parts[1]) + (parts[2] + parts[3])) + (
        (parts[4] + parts[5]) + parts[6])
    sa2 = _sigmoid(conv)

    tmp = jnp.dot(selht_ref[...], sa2, preferred_element_type=jnp.float32)
    sacol = jnp.sum(tmp * wm_ref[:, 0:w], axis=1, keepdims=True)

    o_ref[g] = y * sacol


def _cbam_kernel(x_ref, w_ref, b_ref, xw_ref, sy_ref, selh_ref, selht_ref,
                 wm_ref, o_ref, *, inv_m, inv_c, h, w, gsz):
    for g in range(gsz):
        _one_batch(x_ref, w_ref, b_ref, xw_ref, sy_ref, selh_ref,
                   selht_ref, wm_ref, o_ref, g, inv_m, inv_c, h, w)


def _build_consts(w_sa, h, w):
    f32 = jnp.float32
    w3 = w_sa.reshape(2, 7, 7)
    pw = jnp.arange(w, dtype=jnp.int32)
    dx = pw[:, None] - pw[None, :] + 3
    band = (dx[None] == jnp.arange(7, dtype=jnp.int32)[:, None, None])
    bandf = band.astype(f32)
    xw = jnp.einsum('kij,jab->kiab', w3, bandf)
    xwcat = xw.transpose(0, 2, 1, 3).reshape(2 * w, 7 * w)

    hy = jnp.arange(h, dtype=jnp.int32)
    dy = hy[:, None] - hy[None, :] + 3
    sy = (dy[None] == jnp.arange(7, dtype=jnp.int32)[:, None, None])
    syf = sy.transpose(0, 2, 1).astype(f32).reshape(7 * h, h)

    p = jnp.arange(h * w, dtype=jnp.int32)
    selht = (p[:, None] // w == hy[None, :]).astype(f32)
    selh = selht.T
    w2 = jnp.arange(2 * w, dtype=jnp.int32)
    wm = (p[:, None] % w == w2[None, :] % w).astype(f32)
    return xwcat, syf, selh, selht, wm


def kernel(x, w_fc, b_fc, w_sa):
    B, C, H, W = x.shape
    M = H * W
    xh = jnp.transpose(x, (0, 2, 3, 1)).reshape(B, M, C)
    xwcat, syf, selh, selht, wm = _build_consts(w_sa, H, W)

    G = 4
    while G > 1 and B % (2 * G):
        G //= 2
    body = functools.partial(
        _cbam_kernel, inv_m=1.0 / float(M), inv_c=1.0 / float(C), h=H, w=W,
        gsz=G)
    nb = B // (2 * G)
    cspec = lambda shape: pl.BlockSpec(shape, lambda ci, t: tuple(0 for _ in shape))
    outh = pl.pallas_call(
        body,
        out_shape=jax.ShapeDtypeStruct((B, M, C), x.dtype),
        grid=(2, nb),
        in_specs=[
            pl.BlockSpec((G, M, C), lambda ci, t: (ci * nb + t, 0, 0)),
            cspec((C, C)),
            cspec((1, C)),
            cspec((2 * W, 7 * W)),
            cspec((7 * H, H)),
            cspec((H, M)),
            cspec((M, H)),
            cspec((M, 2 * W)),
        ],
        out_specs=pl.BlockSpec((G, M, C), lambda ci, t: (ci * nb + t, 0, 0)),
        compiler_params=pltpu.CompilerParams(
            dimension_semantics=("parallel", "arbitrary"),
            vmem_limit_bytes=_VMEM_LIMIT),
    )(xh, w_fc, b_fc, xwcat, syf, selh, selht, wm)
    return jnp.transpose(outh.reshape(B, H, W, C), (0, 3, 1, 2))
